# baseline (speedup 1.0000x reference)
import functools

import jax
import jax.numpy as jnp
from jax import lax
from jax.experimental import pallas as pl
from jax.experimental.pallas import tpu as pltpu
from jax.experimental.pallas import tpu_sc as plsc

_VOCAB = 100000
_MAXLEN = 200
_DIM = 32
_BATCH = 4096

_NC = 2
_NS = 16
_NW = _NC * _NS
_BW = _BATCH // _NW
_LC = 4
_TOK = _LC * _BW
_NCHUNK = _MAXLEN // _LC


def _embed_kernel(x_hbm, tok_hbm, pos_hbm, out_hbm,
                  xall_v, idx_v, g_v, t_v, posc_v, gsems, wsems):
    wid = lax.axis_index("s") * _NC + lax.axis_index("c")
    b0 = wid * _BW

    pltpu.sync_copy(x_hbm.at[pl.ds(b0, _BW)],
                    xall_v.at[:, pl.ds(0, _MAXLEN)])
    pltpu.sync_copy(pos_hbm, posc_v)

    iota = lax.iota(jnp.int32, 16)
    rows16 = [iota + 16 * j for j in range(_BW // 16)]

    def prep_idx(g, buf):
        l0 = g * _LC
        for lc in range(_LC):
            col = jnp.full((16,), l0 + lc, jnp.int32)
            for j in range(_BW // 16):
                v = plsc.load_gather(xall_v, [rows16[j], col])
                idx_v[buf, pl.ds(lc * _BW + 16 * j, 16)] = v

    def gather(buf):
        return pltpu.make_async_copy(
            tok_hbm.at[idx_v.at[buf]], g_v.at[buf], gsems.at[buf])

    def writeback(g, buf, dt):
        return pltpu.make_async_copy(
            t_v.at[buf, :, pl.ds(8 * dt, 8), pl.ds(0, _BW)],
            out_hbm.at[pl.ds(g * _LC, _LC), dt, wid],
            wsems.at[buf])

    prep_idx(0, 0)
    gather(0).start()

    def step(g, buf):
        nbuf = 1 - buf
        gather(buf).wait()

        @pl.when(g + 1 < _NCHUNK)
        def _():
            prep_idx(g + 1, nbuf)

        @pl.when(g >= 2)
        def _():
            for dt in range(_DIM // 8):
                writeback(g - 2, buf, dt).wait()

        @pl.when(g + 1 < _NCHUNK)
        def _():
            gather(nbuf).start()

        for lc in range(_LC):
            l = g * _LC + lc
            p0 = posc_v[l, pl.ds(0, 16)]
            p1 = posc_v[l, pl.ds(16, 16)]
            clc = jnp.full((16,), lc, jnp.int32)

            @plsc.parallel_loop(0, _BW, 1, unroll=4)
            def _(b):
                t = lc * _BW + b
                cb = jnp.full((16,), b, jnp.int32)
                v0 = g_v[buf, t, pl.ds(0, 16)] + p0
                v1 = g_v[buf, t, pl.ds(16, 16)] + p1
                plsc.store_scatter(t_v.at[buf], [clc, iota, cb], v0)
                plsc.store_scatter(t_v.at[buf], [clc, iota + 16, cb], v1)

        for dt in range(_DIM // 8):
            writeback(g, buf, dt).start()

    def chunk_body(h, carry):
        step(2 * h, 0)
        step(2 * h + 1, 1)
        return carry

    lax.fori_loop(0, _NCHUNK // 2, chunk_body, 0)

    for g in (_NCHUNK - 2, _NCHUNK - 1):
        for dt in range(_DIM // 8):
            writeback(g, g % 2, dt).wait()


@functools.partial(jax.jit, static_argnames=())
def kernel(x, token_table, pos_table):
    b, l = x.shape
    xi = x.astype(jnp.int32)
    mesh = plsc.VectorSubcoreMesh(core_axis_name="c", subcore_axis_name="s")
    run = functools.partial(
        pl.kernel,
        mesh=mesh,
        compiler_params=pltpu.CompilerParams(use_tc_tiling_on_sc=False,
                                             needs_layout_passes=False),
        out_type=jax.ShapeDtypeStruct((l, _DIM // 8, b // 128, 8, 128),
                                      jnp.float32),
        scratch_types=[
            pltpu.VMEM((_BW, _MAXLEN + 1), jnp.int32),
            pltpu.VMEM((2, _TOK), jnp.int32),
            pltpu.VMEM((2, _TOK, _DIM), jnp.float32),
            pltpu.VMEM((2, _LC, _DIM, _BW + 1), jnp.float32),
            pltpu.VMEM((_MAXLEN, _DIM), jnp.float32),
            pltpu.SemaphoreType.DMA((2,)),
            pltpu.SemaphoreType.DMA((2,)),
        ],
    )(_embed_kernel)
    out = run(xi, token_table, pos_table)
    return out.transpose(2, 4, 0, 1, 3).reshape(b, l, _DIM)


# speedup vs baseline: 1.0188x; 1.0188x over previous
import functools

import jax
import jax.numpy as jnp
from jax import lax
from jax.experimental import pallas as pl
from jax.experimental.pallas import tpu as pltpu
from jax.experimental.pallas import tpu_sc as plsc

_VOCAB = 100000
_MAXLEN = 200
_DIM = 32
_BATCH = 4096

_NC = 2
_NS = 16
_NW = _NC * _NS
_BW = _BATCH // _NW
_LC = 4
_TOK = _LC * _BW
_NCHUNK = _MAXLEN // _LC


def _embed_kernel(x_hbm, tok_hbm, pos_hbm, out_hbm,
                  xall_v, idx_v, g_v, t_v, posc_v, gsems, wsems):
    wid = lax.axis_index("s") * _NC + lax.axis_index("c")
    b0 = wid * _BW

    pltpu.sync_copy(x_hbm.at[pl.ds(b0 * _MAXLEN, _BW * _MAXLEN)], xall_v)
    pltpu.sync_copy(pos_hbm, posc_v)

    iota = lax.iota(jnp.int32, 16)

    def prep_idx(g, buf):
        l0 = g * _LC
        for lc in range(_LC):
            for j in range(_BW // 16):
                flat = (iota + 16 * j) * _MAXLEN + (l0 + lc)
                v = plsc.load_gather(xall_v, [flat])
                idx_v[buf, pl.ds(lc * _BW + 16 * j, 16)] = v

    def gather(buf):
        return pltpu.make_async_copy(
            tok_hbm.at[idx_v.at[buf]], g_v.at[buf], gsems.at[buf])

    def writeback(g, buf, dt):
        return pltpu.make_async_copy(
            t_v.at[buf, :, pl.ds(8 * dt, 8), pl.ds(0, _BW)],
            out_hbm.at[pl.ds(g * _LC, _LC), dt, wid],
            wsems.at[buf])

    prep_idx(0, 0)
    gather(0).start()

    def step(g, buf):
        nbuf = 1 - buf
        gather(buf).wait()

        @pl.when(g + 1 < _NCHUNK)
        def _():
            prep_idx(g + 1, nbuf)

        @pl.when(g >= 2)
        def _():
            for dt in range(_DIM // 8):
                writeback(g - 2, buf, dt).wait()

        @pl.when(g + 1 < _NCHUNK)
        def _():
            gather(nbuf).start()

        for lc in range(_LC):
            l = g * _LC + lc
            p0 = posc_v[l, pl.ds(0, 16)]
            p1 = posc_v[l, pl.ds(16, 16)]
            clc = jnp.full((16,), lc, jnp.int32)

            @plsc.parallel_loop(0, _BW, 1, unroll=4)
            def _(b):
                t = lc * _BW + b
                cb = jnp.full((16,), b, jnp.int32)
                v0 = g_v[buf, t, pl.ds(0, 16)] + p0
                v1 = g_v[buf, t, pl.ds(16, 16)] + p1
                plsc.store_scatter(t_v.at[buf], [clc, iota, cb], v0)
                plsc.store_scatter(t_v.at[buf], [clc, iota + 16, cb], v1)

        for dt in range(_DIM // 8):
            writeback(g, buf, dt).start()

    def chunk_body(h, carry):
        step(2 * h, 0)
        step(2 * h + 1, 1)
        return carry

    lax.fori_loop(0, _NCHUNK // 2, chunk_body, 0)

    for g in (_NCHUNK - 2, _NCHUNK - 1):
        for dt in range(_DIM // 8):
            writeback(g, g % 2, dt).wait()


@functools.partial(jax.jit, static_argnames=())
def kernel(x, token_table, pos_table):
    b, l = x.shape
    xi = x.reshape(-1).astype(jnp.int32)
    mesh = plsc.VectorSubcoreMesh(core_axis_name="c", subcore_axis_name="s")
    run = functools.partial(
        pl.kernel,
        mesh=mesh,
        compiler_params=pltpu.CompilerParams(use_tc_tiling_on_sc=False,
                                             needs_layout_passes=False),
        out_type=jax.ShapeDtypeStruct((l, _DIM // 8, b // 128, 8, 128),
                                      jnp.float32),
        scratch_types=[
            pltpu.VMEM((_BW * _MAXLEN,), jnp.int32),
            pltpu.VMEM((2, _TOK), jnp.int32),
            pltpu.VMEM((2, _TOK, _DIM), jnp.float32),
            pltpu.VMEM((2, _LC, _DIM, _BW + 1), jnp.float32),
            pltpu.VMEM((_MAXLEN, _DIM), jnp.float32),
            pltpu.SemaphoreType.DMA((2,)),
            pltpu.SemaphoreType.DMA((2,)),
        ],
    )(_embed_kernel)
    out = run(xi, token_table, pos_table)
    return out.transpose(2, 4, 0, 1, 3).reshape(b, l, _DIM)
